# Initial kernel scaffold; baseline (speedup 1.0000x reference)
import functools
import math

import jax
import jax.numpy as jnp
from jax import lax
from jax.experimental import pallas as pl
from jax.experimental.pallas import tpu as pltpu
from jax.experimental.pallas import tpu_sc as plsc

N = 10000
E = 320000
D = 128
ALPHA = 0.5
BETAS = (math.log(2.0), math.log(1.5))

NC = 2
NS = 16
NW = NC * NS
CHUNK = 128
K = -(-E // (NW * CHUNK))
E_PAD = NW * K * CHUNK
NROWS = N + NS

_mesh = plsc.VectorSubcoreMesh(core_axis_name="c", subcore_axis_name="s")


@functools.partial(
    pl.kernel,
    mesh=_mesh,
    out_type=jax.ShapeDtypeStruct((NC, NROWS, D), jnp.float32),
    scratch_types=[
        pltpu.VMEM((CHUNK,), jnp.int32),
        pltpu.VMEM((CHUNK,), jnp.int32),
        pltpu.VMEM((CHUNK, D), jnp.float32),
        pltpu.VMEM_SHARED((NROWS, D), jnp.float32),
        pltpu.SemaphoreType.DMA,
    ],
)
def _sc_agg(h_hbm, src_hbm, dst_hbm, zeros_hbm, out_hbm,
            src_v, dst_v, rows_v, acc_sh, sem):
    cid = lax.axis_index("c")
    sid = lax.axis_index("s")
    wid = cid * NS + sid
    zrows = NROWS // NS
    pltpu.sync_copy(zeros_hbm.at[pl.ds(sid * zrows, zrows)],
                    acc_sh.at[pl.ds(sid * zrows, zrows)])
    plsc.subcore_barrier()

    base = wid * K * CHUNK

    @pl.loop(0, K)
    def _(i):
        off = base + i * CHUNK
        pltpu.sync_copy(src_hbm.at[pl.ds(off, CHUNK)], src_v)
        pltpu.sync_copy(dst_hbm.at[pl.ds(off, CHUNK)], dst_v)
        pltpu.async_copy(h_hbm.at[src_v], rows_v, sem).wait()
        pltpu.sync_copy(rows_v, acc_sh.at[dst_v], add=True)

    plsc.subcore_barrier()
    pltpu.sync_copy(acc_sh.at[pl.ds(sid * zrows, zrows)],
                    out_hbm.at[cid, pl.ds(sid * zrows, zrows)])


_ROWBLK = 1000
_GRID = N // _ROWBLK


def _rowspec():
    return pl.BlockSpec((_ROWBLK, D), lambda i: (i, 0))


def _wspec():
    return pl.BlockSpec((D, D), lambda i: (0, 0))


def _mm_body(x_ref, w_ref, o_ref):
    o_ref[...] = jnp.dot(x_ref[...], w_ref[...],
                         preferred_element_type=jnp.float32)


def _tc_matmul(x, w):
    return pl.pallas_call(
        _mm_body,
        grid=(_GRID,),
        in_specs=[_rowspec(), _wspec()],
        out_specs=_rowspec(),
        out_shape=jax.ShapeDtypeStruct((N, D), jnp.float32),
    )(x, w)


def _in_body(p0_ref, p1_ref, b_ref, o_ref):
    o_ref[...] = jnp.maximum(p0_ref[...] + p1_ref[...] + b_ref[...], 0.0)


def _tc_combine_in(p0, p1, b):
    return pl.pallas_call(
        _in_body,
        grid=(_GRID,),
        in_specs=[_rowspec(), _rowspec(),
                  pl.BlockSpec((1, D), lambda i: (0, 0))],
        out_specs=_rowspec(),
        out_shape=jax.ShapeDtypeStruct((N, D), jnp.float32),
    )(p0, p1, b.reshape(1, D))


def _layer_body(p0_ref, p1_ref, x0_ref, w_ref, o_ref, *, beta):
    agg = p0_ref[...] + p1_ref[...]
    out = agg * (1.0 - ALPHA) + ALPHA * x0_ref[...]
    h = (1.0 - beta) * out + beta * jnp.dot(
        out, w_ref[...], preferred_element_type=jnp.float32)
    o_ref[...] = jnp.maximum(h, 0.0)


def _tc_layer(p0, p1, x0, w, beta):
    return pl.pallas_call(
        functools.partial(_layer_body, beta=beta),
        grid=(_GRID,),
        in_specs=[_rowspec(), _rowspec(), _rowspec(), _wspec()],
        out_specs=_rowspec(),
        out_shape=jax.ShapeDtypeStruct((N, D), jnp.float32),
    )(p0, p1, x0, w)


def _layer_out_body(p0_ref, p1_ref, x0_ref, w_ref, wo_ref, o_ref, *, beta):
    agg = p0_ref[...] + p1_ref[...]
    out = agg * (1.0 - ALPHA) + ALPHA * x0_ref[...]
    h = (1.0 - beta) * out + beta * jnp.dot(
        out, w_ref[...], preferred_element_type=jnp.float32)
    h = jnp.maximum(h, 0.0)
    o_ref[...] = jnp.dot(h, wo_ref[...], preferred_element_type=jnp.float32)


def _tc_layer_out(p0, p1, x0, w, beta, w_out):
    return pl.pallas_call(
        functools.partial(_layer_out_body, beta=beta),
        grid=(_GRID,),
        in_specs=[_rowspec(), _rowspec(), _rowspec(), _wspec(), _wspec()],
        out_specs=_rowspec(),
        out_shape=jax.ShapeDtypeStruct((N, D), jnp.float32),
    )(p0, p1, x0, w, w_out)


def _fin_body(p0_ref, p1_ref, b_ref, o_ref):
    o_ref[...] = p0_ref[...] + p1_ref[...] + b_ref[...]


def _tc_final(p0, p1, b):
    return pl.pallas_call(
        _fin_body,
        grid=(_GRID,),
        in_specs=[_rowspec(), _rowspec(),
                  pl.BlockSpec((1, D), lambda i: (0, 0))],
        out_specs=_rowspec(),
        out_shape=jax.ShapeDtypeStruct((N, D), jnp.float32),
    )(p0, p1, b.reshape(1, D))


def _agg(h, src_p, dst_p, zeros_hbm):
    p = _sc_agg(h, src_p, dst_p, zeros_hbm)
    return p[0, :N], p[1, :N]


def kernel(x, edge_index, W_in, b_in, W_layers, W_out, b_out):
    pad = E_PAD - E
    src_p = jnp.concatenate(
        [edge_index[0], jnp.zeros((pad,), jnp.int32)])
    dst_p = jnp.concatenate(
        [edge_index[1], jnp.full((pad,), N, jnp.int32)])
    zeros_hbm = jnp.zeros((NROWS, D), jnp.float32)

    h = _tc_matmul(x, W_in)
    p0, p1 = _agg(h, src_p, dst_p, zeros_hbm)
    x0 = _tc_combine_in(p0, p1, b_in)
    p0, p1 = _agg(x0, src_p, dst_p, zeros_hbm)
    h = _tc_layer(p0, p1, x0, W_layers[0], BETAS[0])
    p0, p1 = _agg(h, src_p, dst_p, zeros_hbm)
    h = _tc_layer_out(p0, p1, x0, W_layers[1], BETAS[1], W_out)
    p0, p1 = _agg(h, src_p, dst_p, zeros_hbm)
    return _tc_final(p0, p1, b_out)


# speedup vs baseline: 3.5099x; 3.5099x over previous
import functools
import math

import jax
import jax.numpy as jnp
from jax import lax
from jax.experimental import pallas as pl
from jax.experimental.pallas import tpu as pltpu
from jax.experimental.pallas import tpu_sc as plsc

N = 10000
E = 320000
D = 128
ALPHA = 0.5
BETAS = (math.log(2.0), math.log(1.5))

NC = 2
NS = 16
NW = NC * NS
CHUNK = 128
K = -(-E // (NW * CHUNK))
E_PAD = NW * K * CHUNK
NROWS = -(-(N + 1) // 128) * 128

_mesh = plsc.VectorSubcoreMesh(core_axis_name="c", subcore_axis_name="s")


@functools.partial(
    pl.kernel,
    mesh=_mesh,
    out_type=jax.ShapeDtypeStruct((NC, NROWS, D), jnp.float32),
    scratch_types=[
        pltpu.VMEM((CHUNK,), jnp.int32),
        pltpu.VMEM((CHUNK,), jnp.int32),
        pltpu.VMEM((CHUNK, D), jnp.float32),
        pltpu.VMEM_SHARED((NROWS, D), jnp.float32),
        pltpu.SemaphoreType.DMA,
    ],
)
def _sc_agg(h_hbm, src_hbm, dst_hbm, zeros_hbm, out_hbm,
            src_v, dst_v, rows_v, acc_sh, sem):
    cid = lax.axis_index("c")
    sid = lax.axis_index("s")
    wid = cid * NS + sid
    zrows = NROWS // NS
    pltpu.sync_copy(zeros_hbm.at[pl.ds(sid * zrows, zrows)],
                    acc_sh.at[pl.ds(sid * zrows, zrows)])
    plsc.subcore_barrier()

    base = wid * K * CHUNK

    @pl.loop(0, K)
    def _(i):
        off = base + i * CHUNK
        pltpu.sync_copy(src_hbm.at[pl.ds(off, CHUNK)], src_v)
        pltpu.sync_copy(dst_hbm.at[pl.ds(off, CHUNK)], dst_v)
        pltpu.async_copy(h_hbm.at[src_v], rows_v, sem).wait()
        pltpu.sync_copy(rows_v, acc_sh.at[dst_v], add=True)

    plsc.subcore_barrier()
    pltpu.sync_copy(acc_sh.at[pl.ds(sid * zrows, zrows)],
                    out_hbm.at[cid, pl.ds(sid * zrows, zrows)])


_ROWBLK = 1000
_GRID = N // _ROWBLK


def _rowspec():
    return pl.BlockSpec((_ROWBLK, D), lambda i: (i, 0))


def _wspec():
    return pl.BlockSpec((D, D), lambda i: (0, 0))


def _mm_body(x_ref, w_ref, o_ref):
    o_ref[...] = jnp.dot(x_ref[...], w_ref[...],
                         preferred_element_type=jnp.float32)


def _tc_matmul(x, w):
    return pl.pallas_call(
        _mm_body,
        grid=(_GRID,),
        in_specs=[_rowspec(), _wspec()],
        out_specs=_rowspec(),
        out_shape=jax.ShapeDtypeStruct((N, D), jnp.float32),
    )(x, w)


def _in_body(p0_ref, p1_ref, b_ref, o_ref):
    o_ref[...] = jnp.maximum(p0_ref[...] + p1_ref[...] + b_ref[...], 0.0)


def _tc_combine_in(p0, p1, b):
    return pl.pallas_call(
        _in_body,
        grid=(_GRID,),
        in_specs=[_rowspec(), _rowspec(),
                  pl.BlockSpec((1, D), lambda i: (0, 0))],
        out_specs=_rowspec(),
        out_shape=jax.ShapeDtypeStruct((N, D), jnp.float32),
    )(p0, p1, b.reshape(1, D))


def _layer_body(p0_ref, p1_ref, x0_ref, w_ref, o_ref, *, beta):
    agg = p0_ref[...] + p1_ref[...]
    out = agg * (1.0 - ALPHA) + ALPHA * x0_ref[...]
    h = (1.0 - beta) * out + beta * jnp.dot(
        out, w_ref[...], preferred_element_type=jnp.float32)
    o_ref[...] = jnp.maximum(h, 0.0)


def _tc_layer(p0, p1, x0, w, beta):
    return pl.pallas_call(
        functools.partial(_layer_body, beta=beta),
        grid=(_GRID,),
        in_specs=[_rowspec(), _rowspec(), _rowspec(), _wspec()],
        out_specs=_rowspec(),
        out_shape=jax.ShapeDtypeStruct((N, D), jnp.float32),
    )(p0, p1, x0, w)


def _layer_out_body(p0_ref, p1_ref, x0_ref, w_ref, wo_ref, o_ref, *, beta):
    agg = p0_ref[...] + p1_ref[...]
    out = agg * (1.0 - ALPHA) + ALPHA * x0_ref[...]
    h = (1.0 - beta) * out + beta * jnp.dot(
        out, w_ref[...], preferred_element_type=jnp.float32)
    h = jnp.maximum(h, 0.0)
    o_ref[...] = jnp.dot(h, wo_ref[...], preferred_element_type=jnp.float32)


def _tc_layer_out(p0, p1, x0, w, beta, w_out):
    return pl.pallas_call(
        functools.partial(_layer_out_body, beta=beta),
        grid=(_GRID,),
        in_specs=[_rowspec(), _rowspec(), _rowspec(), _wspec(), _wspec()],
        out_specs=_rowspec(),
        out_shape=jax.ShapeDtypeStruct((N, D), jnp.float32),
    )(p0, p1, x0, w, w_out)


def _fin_body(p0_ref, p1_ref, b_ref, o_ref):
    o_ref[...] = p0_ref[...] + p1_ref[...] + b_ref[...]


def _tc_final(p0, p1, b):
    return pl.pallas_call(
        _fin_body,
        grid=(_GRID,),
        in_specs=[_rowspec(), _rowspec(),
                  pl.BlockSpec((1, D), lambda i: (0, 0))],
        out_specs=_rowspec(),
        out_shape=jax.ShapeDtypeStruct((N, D), jnp.float32),
    )(p0, p1, b.reshape(1, D))


def _agg(h, src_p, dst_p, zeros_hbm):
    p = _sc_agg(h, src_p, dst_p, zeros_hbm)
    return p[0, :N], p[1, :N]


def kernel(x, edge_index, W_in, b_in, W_layers, W_out, b_out):
    pad = E_PAD - E
    src_p = jnp.concatenate(
        [edge_index[0], jnp.zeros((pad,), jnp.int32)])
    dst_p = jnp.concatenate(
        [edge_index[1], jnp.full((pad,), N, jnp.int32)])
    zeros_hbm = jnp.zeros((NROWS, D), jnp.float32)

    h = _tc_matmul(x, W_in)
    p0, p1 = _agg(h, src_p, dst_p, zeros_hbm)
    x0 = _tc_combine_in(p0, p1, b_in)
    p0, p1 = _agg(x0, src_p, dst_p, zeros_hbm)
    h = _tc_layer(p0, p1, x0, W_layers[0], BETAS[0])
    p0, p1 = _agg(h, src_p, dst_p, zeros_hbm)
    h = _tc_layer_out(p0, p1, x0, W_layers[1], BETAS[1], W_out)
    p0, p1 = _agg(h, src_p, dst_p, zeros_hbm)
    return _tc_final(p0, p1, b_out)
